# baseline (speedup 1.0000x reference)
import functools

import jax
import jax.numpy as jnp
import numpy as np
from jax import lax
from jax.experimental import pallas as pl
from jax.experimental.pallas import tpu as pltpu
from jax.experimental.pallas import tpu_sc as plsc

N = 10000
N_PAD = 10240
E = 160000
EPS = 1e-8
NORM = 1.0 / np.sqrt(16.0)
C_OLD = 1.0 / np.sqrt(1.25)
C_NEW = 0.5 * C_OLD
POST_SCALE = C_NEW * NORM

NC = 2
NS = 16
NW = NC * NS

_cols = np.arange(352)
_grp = np.where(_cols < 192, _cols // 3, 64 + (_cols - 192) // 5)
_EXP_NP = (np.arange(96)[:, None] == _grp[None, :]).astype(np.float32)


def _node_prep_body(nf_ref, g_ref, b_ref, w1_ref, w3_ref, a_ref, bb_ref,
                    res_ref):
    x = nf_ref[...]
    g = g_ref[...]
    beta = b_ref[...]
    scal = x[:, :128]
    mu = jnp.mean(scal, axis=-1, keepdims=True)
    var = jnp.mean((scal - mu) * (scal - mu), axis=-1, keepdims=True)
    scal_n = (scal - mu) * lax.rsqrt(var + EPS) * g[:, :128] + beta
    b1 = x[:, 128:320]
    b2 = x[:, 320:480]
    bal = 0.5 * (jnp.mean(b1 * b1, axis=-1) + jnp.mean(b2 * b2, axis=-1))
    rest_n = x[:, 128:480] * lax.rsqrt(bal + EPS)[:, None] * g[:, 128:]
    nn = jnp.concatenate([scal_n, rest_n], axis=-1).astype(jnp.bfloat16)
    a_ref[:, :288] = _pack576(
        jnp.dot(nn, w1_ref[...], preferred_element_type=jnp.float32))
    a_ref[:, 288:] = jnp.zeros_like(a_ref[:, 288:])
    bb_ref[:, :288] = _pack576(
        jnp.dot(nn, w3_ref[...], preferred_element_type=jnp.float32))
    bb_ref[:, 288:] = jnp.zeros_like(bb_ref[:, 288:])
    res_ref[:, :480] = C_OLD * x
    res_ref[:, 480:] = jnp.zeros_like(res_ref[:, 480:])


def _pack576(x):
    def rne16(v):
        u = lax.bitcast_convert_type(v, jnp.uint32)
        return (u + jnp.uint32(0x7FFF) + ((u >> 16) & jnp.uint32(1))) >> 16

    w = rne16(x[:, :288]) | (rne16(x[:, 288:]) << 16)
    return lax.bitcast_convert_type(w, jnp.int32)


def _unpack576(w):
    lo = lax.bitcast_convert_type(w << 16, jnp.float32)
    hi = lax.bitcast_convert_type((w >> 16) << 16, jnp.float32)
    return lo, hi


def _node_prep(nf, gamma_n, beta_n, w1, w3):
    blk = 1000
    grid = N // blk
    return pl.pallas_call(
        _node_prep_body,
        grid=(grid,),
        in_specs=[
            pl.BlockSpec((blk, 480), lambda i: (i, 0)),
            pl.BlockSpec((1, 480), lambda i: (0, 0)),
            pl.BlockSpec((1, 128), lambda i: (0, 0)),
            pl.BlockSpec((480, 576), lambda i: (0, 0)),
            pl.BlockSpec((480, 576), lambda i: (0, 0)),
        ],
        out_specs=[
            pl.BlockSpec((blk, 384), lambda i: (i, 0)),
            pl.BlockSpec((blk, 384), lambda i: (i, 0)),
            pl.BlockSpec((blk, 512), lambda i: (i, 0)),
        ],
        out_shape=[
            jax.ShapeDtypeStruct((N, 384), jnp.int32),
            jax.ShapeDtypeStruct((N, 384), jnp.int32),
            jax.ShapeDtypeStruct((N_PAD, 512), jnp.float32),
        ],
    )(nf, gamma_n, beta_n, w1, w3)


_GC = 32
_G_EW = E // NW
_G_FULL = _G_EW // _GC
_G_TAIL = _G_EW - _G_FULL * _GC
_G_NCHP = _G_FULL + 1


def _sc_gather(a, b, ec3d, en3d):
    mesh = plsc.VectorSubcoreMesh(core_axis_name="c", subcore_axis_name="s")

    @functools.partial(
        pl.kernel,
        out_type=[
            jax.ShapeDtypeStruct((E, 384), jnp.int32),
            jax.ShapeDtypeStruct((E, 384), jnp.int32),
        ],
        mesh=mesh,
        scratch_types=[
            pltpu.VMEM((_G_NCHP, _GC), jnp.int32),
            pltpu.VMEM((_G_NCHP, _GC), jnp.int32),
            pltpu.VMEM((_GC, 384), jnp.int32),
            pltpu.VMEM((_GC, 384), jnp.int32),
            pltpu.VMEM((_GC, 384), jnp.int32),
            pltpu.VMEM((_GC, 384), jnp.int32),
            pltpu.VMEM((_GC, 384), jnp.int32),
            pltpu.VMEM((_GC, 384), jnp.int32),
            pltpu.SemaphoreType.DMA,
            pltpu.SemaphoreType.DMA,
            pltpu.SemaphoreType.DMA,
            pltpu.SemaphoreType.DMA,
            pltpu.SemaphoreType.DMA,
            pltpu.SemaphoreType.DMA,
        ],
    )
    def k(a_hbm, b_hbm, ec_hbm, en_hbm, ga_hbm, gb_hbm,
          ec_v, en_v, bufa0, bufb0, bufa1, bufb1, bufa2, bufb2,
          sg0, sg1, sg2, sw0, sw1, sw2):
        wid = lax.axis_index("s") * NC + lax.axis_index("c")
        pltpu.sync_copy(ec_hbm.at[wid], ec_v)
        pltpu.sync_copy(en_hbm.at[wid], en_v)
        base = wid * _G_EW
        bufs = ((bufa0, bufb0, sg0, sw0), (bufa1, bufb1, sg1, sw1),
                (bufa2, bufb2, sg2, sw2))

        def start_gather(i, p):
            ba, bb, sg, _ = bufs[p]
            pltpu.async_copy(a_hbm.at[ec_v.at[i]], ba, sg)
            pltpu.async_copy(b_hbm.at[en_v.at[i]], bb, sg)

        def finish(i, p):
            ba, bb, sg, sw = bufs[p]
            pltpu.make_async_copy(a_hbm.at[ec_v.at[i]], ba, sg).wait()
            pltpu.make_async_copy(b_hbm.at[en_v.at[i]], bb, sg).wait()
            pltpu.async_copy(
                ba, ga_hbm.at[pl.ds(base + i * _GC, _GC)], sw)
            pltpu.async_copy(
                bb, gb_hbm.at[pl.ds(base + i * _GC, _GC)], sw)

        def wait_write(i, p):
            ba, bb, _, sw = bufs[p]
            pltpu.make_async_copy(
                ba, ga_hbm.at[pl.ds(base + i * _GC, _GC)], sw).wait()
            pltpu.make_async_copy(
                bb, gb_hbm.at[pl.ds(base + i * _GC, _GC)], sw).wait()

        start_gather(0, 0)
        start_gather(1, 1)

        def macro_step(ii, carry):
            i = ii * 3

            def sub(q, p):
                c = i + q
                finish(c, p)

                @pl.when(c >= 1)
                def _():
                    wait_write(c - 1, (p + 2) % 3)

                @pl.when(c + 2 < _G_FULL)
                def _():
                    start_gather(c + 2, (p + 2) % 3)

            sub(0, 0)
            sub(1, 1)
            sub(2, 2)
            return carry

        lax.fori_loop(0, _G_FULL // 3, macro_step, 0)

        ba, bb, sg, sw = bufs[0]
        pltpu.async_copy(a_hbm.at[ec_v.at[_G_FULL]], ba, sg)
        pltpu.async_copy(b_hbm.at[en_v.at[_G_FULL]], bb, sg)
        pltpu.make_async_copy(a_hbm.at[ec_v.at[_G_FULL]], ba, sg).wait()
        pltpu.make_async_copy(b_hbm.at[en_v.at[_G_FULL]], bb, sg).wait()
        pltpu.sync_copy(
            ba.at[pl.ds(0, _G_TAIL)],
            ga_hbm.at[pl.ds(base + _G_FULL * _GC, _G_TAIL)])
        pltpu.sync_copy(
            bb.at[pl.ds(0, _G_TAIL)],
            gb_hbm.at[pl.ds(base + _G_FULL * _GC, _G_TAIL)])
        wait_write(_G_FULL - 1, 2)

    return k(a, b, ec3d, en3d)


def _main_body(ga_ref, gb_ref, ef_ref, lat_ref, ge_ref, be_ref, w2_ref,
               btp_ref, wpost_ref, bvec_ref, wenv_ref, benv_ref, exp_ref,
               em_ref, wt_ref):
    ef = ef_ref[...]
    g = ge_ref[...]
    beta = be_ref[...]
    scal = ef[:, :64]
    mu = jnp.mean(scal, axis=-1, keepdims=True)
    var = jnp.mean((scal - mu) * (scal - mu), axis=-1, keepdims=True)
    scal_n = (scal - mu) * lax.rsqrt(var + EPS) * g[:, :64] + beta
    b1 = ef[:, 64:160]
    b2 = ef[:, 160:240]
    bal = 0.5 * (jnp.mean(b1 * b1, axis=-1) + jnp.mean(b2 * b2, axis=-1))
    rest_n = ef[:, 64:240] * lax.rsqrt(bal + EPS)[:, None] * g[:, 64:]
    ne = jnp.concatenate([scal_n, rest_n], axis=-1)

    a_lo, a_hi = _unpack576(ga_ref[:, :288])
    b_lo, b_hi = _unpack576(gb_ref[:, :288])
    ne_dot = jnp.dot(ne.astype(jnp.bfloat16), w2_ref[...],
                     preferred_element_type=jnp.float32)
    msg_l = a_lo + b_lo + btp_ref[:, :288] + ne_dot[:, :288]
    msg_h = a_hi + b_hi + btp_ref[:, 288:] + ne_dot[:, 288:]

    scal_m = msg_l[:, :128]
    silu = scal_m * jax.nn.sigmoid(scal_m)
    gates = jax.nn.sigmoid(msg_l[:, 128:224]).astype(jnp.bfloat16)
    rest_act_l = msg_l[:, 224:288] * jnp.dot(
        gates, exp_ref[:, :64], preferred_element_type=jnp.float32)
    rest_act_h = msg_h * jnp.dot(
        gates, exp_ref[:, 64:], preferred_element_type=jnp.float32)

    em = (jnp.dot(silu.astype(jnp.bfloat16), wpost_ref[:128, :],
                  preferred_element_type=jnp.float32)
          + jnp.dot(rest_act_l.astype(jnp.bfloat16), wpost_ref[128:192, :],
                    preferred_element_type=jnp.float32)
          + jnp.dot(rest_act_h.astype(jnp.bfloat16), wpost_ref[192:, :],
                    preferred_element_type=jnp.float32)
          + bvec_ref[...])
    em_ref[...] = em

    w = (jnp.dot(lat_ref[...].astype(jnp.bfloat16), wenv_ref[...],
                 preferred_element_type=jnp.float32) + benv_ref[...])
    wexp_rest = jnp.dot(w[:, 128:224].astype(jnp.bfloat16), exp_ref[...],
                        preferred_element_type=jnp.float32)
    wt_ref[:, :128] = (POST_SCALE * em[:, :128]) * w[:, :128]
    wt_ref[:, 128:480] = (POST_SCALE * em[:, 128:480]) * wexp_rest
    wt_ref[:, 480:512] = jnp.zeros_like(wt_ref[:, 480:512])


def _main(ga, gb, ef, lat, gamma_e, beta_e, w2, btp, wpost, bvec, wenv,
          benv, expm):
    blk = 640
    grid = E // blk
    return pl.pallas_call(
        _main_body,
        grid=(grid,),
        in_specs=[
            pl.BlockSpec((blk, 384), lambda i: (i, 0)),
            pl.BlockSpec((blk, 384), lambda i: (i, 0)),
            pl.BlockSpec((blk, 240), lambda i: (i, 0)),
            pl.BlockSpec((blk, 64), lambda i: (i, 0)),
            pl.BlockSpec((1, 240), lambda i: (0, 0)),
            pl.BlockSpec((1, 64), lambda i: (0, 0)),
            pl.BlockSpec((240, 576), lambda i: (0, 0)),
            pl.BlockSpec((1, 576), lambda i: (0, 0)),
            pl.BlockSpec((480, 480), lambda i: (0, 0)),
            pl.BlockSpec((1, 480), lambda i: (0, 0)),
            pl.BlockSpec((64, 224), lambda i: (0, 0)),
            pl.BlockSpec((1, 224), lambda i: (0, 0)),
            pl.BlockSpec((96, 352), lambda i: (0, 0)),
        ],
        out_specs=[
            pl.BlockSpec((blk, 480), lambda i: (i, 0)),
            pl.BlockSpec((blk, 512), lambda i: (i, 0)),
        ],
        out_shape=[
            jax.ShapeDtypeStruct((E, 480), jnp.float32),
            jax.ShapeDtypeStruct((E, 512), jnp.float32),
        ],
    )(ga, gb, ef, lat, gamma_e, beta_e, w2, btp, wpost, bvec, wenv, benv,
      expm)


_SC_EC = 80
_S_EW = E // NS
_S_NCH = _S_EW // _SC_EC
_S_RT = N_PAD // NS


def _sc_scatter(wt, ec3d, res):
    mesh = plsc.VectorSubcoreMesh(core_axis_name="c", subcore_axis_name="s")

    @functools.partial(
        pl.kernel,
        out_type=jax.ShapeDtypeStruct((N_PAD, 512), jnp.float32),
        mesh=mesh,
        scratch_types=[
            pltpu.VMEM((_S_NCH, _SC_EC), jnp.int32),
            pltpu.VMEM((_SC_EC, 128), jnp.float32),
            pltpu.VMEM((_SC_EC, 128), jnp.float32),
            pltpu.VMEM_SHARED((N_PAD, 128), jnp.float32),
            pltpu.SemaphoreType.DMA,
            pltpu.SemaphoreType.DMA,
        ],
    )
    def k(wt_hbm, ec_hbm, res_hbm, out_hbm, ec_v, upd0, upd1, acc, sr0, sr1):
        c = lax.axis_index("c")
        s = lax.axis_index("s")
        pltpu.sync_copy(ec_hbm.at[s], ec_v)
        r0 = s * _S_RT
        upds = ((upd0, sr0), (upd1, sr1))

        for chunk in range(2):
            col0 = (c * 2 + chunk) * 128
            pltpu.sync_copy(
                res_hbm.at[pl.ds(r0, _S_RT), pl.ds(col0, 128)],
                acc.at[pl.ds(r0, _S_RT)])
            plsc.subcore_barrier()

            def start_read(i, p):
                u, sr = upds[p]
                pltpu.async_copy(
                    wt_hbm.at[pl.ds(s * _S_EW + i * _SC_EC, _SC_EC),
                              pl.ds(col0, 128)], u, sr)

            def do_add(i, p):
                u, sr = upds[p]
                pltpu.make_async_copy(
                    wt_hbm.at[pl.ds(s * _S_EW + i * _SC_EC, _SC_EC),
                              pl.ds(col0, 128)], u, sr).wait()
                pltpu.sync_copy(u, acc.at[ec_v.at[i]], add=True)

            start_read(0, 0)
            start_read(1, 1)

            def step(ii, carry):
                i = ii * 2
                do_add(i, 0)

                @pl.when(ii + 1 < _S_NCH // 2)
                def _():
                    start_read(i + 2, 0)
                do_add(i + 1, 1)

                @pl.when(ii + 1 < _S_NCH // 2)
                def _():
                    start_read(i + 3, 1)
                return carry

            lax.fori_loop(0, _S_NCH // 2, step, 0)
            start_read(_S_NCH - 1, 0)
            do_add(_S_NCH - 1, 0)
            plsc.subcore_barrier()

            pltpu.sync_copy(
                acc.at[pl.ds(r0, _S_RT)],
                out_hbm.at[pl.ds(r0, _S_RT), pl.ds(col0, 128)])
            plsc.subcore_barrier()

    return k(wt, ec3d, res)


def _slice_body(i_ref, o_ref):
    o_ref[...] = i_ref[:, :480]


def _slice_out(x):
    blk = 1000
    return pl.pallas_call(
        _slice_body,
        grid=(N // blk,),
        in_specs=[pl.BlockSpec((blk, 512), lambda i: (i, 0))],
        out_specs=pl.BlockSpec((blk, 480), lambda i: (i, 0)),
        out_shape=jax.ShapeDtypeStruct((N, 480), jnp.float32),
    )(x)


def kernel(latents, node_features, edge_features, atom_type, edge_index,
           edge_vector, active_edges, wigner_D_all, gamma_n, beta_n,
           gamma_e, beta_e, W_tp, b_tp, W_post, b_post, W_env, b_env):
    del atom_type, edge_vector, active_edges
    ec = edge_index[0].astype(jnp.int32)
    en = edge_index[1].astype(jnp.int32)

    w1 = W_tp[:480].astype(jnp.bfloat16)
    w2 = W_tp[480:720].astype(jnp.bfloat16)
    w3 = W_tp[720:].astype(jnp.bfloat16)

    a, b, res = _node_prep(node_features, gamma_n.reshape(1, 480),
                           beta_n.reshape(1, 128), w1, w3)

    ec_g = jnp.pad(ec.reshape(NW, _G_EW),
                   ((0, 0), (0, _G_NCHP * _GC - _G_EW)))
    en_g = jnp.pad(en.reshape(NW, _G_EW),
                   ((0, 0), (0, _G_NCHP * _GC - _G_EW)))
    ga, gb = _sc_gather(a, b, ec_g.reshape(NW, _G_NCHP, _GC),
                        en_g.reshape(NW, _G_NCHP, _GC))

    bvec = jnp.concatenate([b_post, jnp.zeros((352,), jnp.float32)])
    em, wt = _main(ga, gb, edge_features, latents,
                   gamma_e.reshape(1, 240), beta_e.reshape(1, 64), w2,
                   b_tp.reshape(1, 576), W_post.astype(jnp.bfloat16),
                   bvec.reshape(1, 480), W_env.astype(jnp.bfloat16),
                   b_env.reshape(1, 224),
                   jnp.asarray(_EXP_NP, dtype=jnp.bfloat16))

    out_pad = _sc_scatter(wt, ec.reshape(NS, _S_NCH, _SC_EC), res)
    out = _slice_out(out_pad)
    return (out, em, wigner_D_all)


# speedup vs baseline: 1.0714x; 1.0714x over previous
import functools

import jax
import jax.numpy as jnp
import numpy as np
from jax import lax
from jax.experimental import pallas as pl
from jax.experimental.pallas import tpu as pltpu
from jax.experimental.pallas import tpu_sc as plsc

N = 10000
N_PAD = 10240
E = 160000
EPS = 1e-8
NORM = 1.0 / np.sqrt(16.0)
C_OLD = 1.0 / np.sqrt(1.25)
C_NEW = 0.5 * C_OLD
POST_SCALE = C_NEW * NORM

NC = 2
NS = 16
NW = NC * NS

_cols = np.arange(352)
_grp = np.where(_cols < 192, _cols // 3, 64 + (_cols - 192) // 5)
_EXP_NP = (np.arange(96)[:, None] == _grp[None, :]).astype(np.float32)


def _node_prep_body(nf_ref, g_ref, b_ref, w1_ref, w3_ref, a_ref, bb_ref,
                    res_ref):
    x = nf_ref[...]
    g = g_ref[...]
    beta = b_ref[...]
    scal = x[:, :128]
    mu = jnp.mean(scal, axis=-1, keepdims=True)
    var = jnp.mean((scal - mu) * (scal - mu), axis=-1, keepdims=True)
    scal_n = (scal - mu) * lax.rsqrt(var + EPS) * g[:, :128] + beta
    b1 = x[:, 128:320]
    b2 = x[:, 320:480]
    bal = 0.5 * (jnp.mean(b1 * b1, axis=-1) + jnp.mean(b2 * b2, axis=-1))
    rest_n = x[:, 128:480] * lax.rsqrt(bal + EPS)[:, None] * g[:, 128:]
    nn = jnp.concatenate([scal_n, rest_n], axis=-1).astype(jnp.bfloat16)
    a_ref[:, :288] = _pack576(
        jnp.dot(nn, w1_ref[...], preferred_element_type=jnp.float32))
    a_ref[:, 288:] = jnp.zeros_like(a_ref[:, 288:])
    bb_ref[:, :288] = _pack576(
        jnp.dot(nn, w3_ref[...], preferred_element_type=jnp.float32))
    bb_ref[:, 288:] = jnp.zeros_like(bb_ref[:, 288:])
    res_ref[:, :480] = C_OLD * x
    res_ref[:, 480:] = jnp.zeros_like(res_ref[:, 480:])


def _pack576(x):
    def rne16(v):
        u = lax.bitcast_convert_type(v, jnp.uint32)
        return (u + jnp.uint32(0x7FFF) + ((u >> 16) & jnp.uint32(1))) >> 16

    w = rne16(x[:, :288]) | (rne16(x[:, 288:]) << 16)
    return lax.bitcast_convert_type(w, jnp.int32)


def _unpack576(w):
    lo = lax.bitcast_convert_type(w << 16, jnp.float32)
    hi = lax.bitcast_convert_type((w >> 16) << 16, jnp.float32)
    return lo, hi


def _node_prep(nf, gamma_n, beta_n, w1, w3):
    blk = 1000
    grid = N // blk
    return pl.pallas_call(
        _node_prep_body,
        grid=(grid,),
        in_specs=[
            pl.BlockSpec((blk, 480), lambda i: (i, 0)),
            pl.BlockSpec((1, 480), lambda i: (0, 0)),
            pl.BlockSpec((1, 128), lambda i: (0, 0)),
            pl.BlockSpec((480, 576), lambda i: (0, 0)),
            pl.BlockSpec((480, 576), lambda i: (0, 0)),
        ],
        out_specs=[
            pl.BlockSpec((blk, 384), lambda i: (i, 0)),
            pl.BlockSpec((blk, 384), lambda i: (i, 0)),
            pl.BlockSpec((blk, 512), lambda i: (i, 0)),
        ],
        out_shape=[
            jax.ShapeDtypeStruct((N, 384), jnp.int32),
            jax.ShapeDtypeStruct((N, 384), jnp.int32),
            jax.ShapeDtypeStruct((N_PAD, 512), jnp.float32),
        ],
    )(nf, gamma_n, beta_n, w1, w3)


_GC = 32
_G_EW = E // NW
_G_FULL = _G_EW // _GC
_G_TAIL = _G_EW - _G_FULL * _GC
_G_NCHP = _G_FULL + 1


def _sc_gather(a, b, ec3d, en3d):
    mesh = plsc.VectorSubcoreMesh(core_axis_name="c", subcore_axis_name="s")

    @functools.partial(
        pl.kernel,
        out_type=[
            jax.ShapeDtypeStruct((E, 384), jnp.int32),
            jax.ShapeDtypeStruct((E, 384), jnp.int32),
        ],
        mesh=mesh,
        scratch_types=[
            pltpu.VMEM((_G_NCHP, _GC), jnp.int32),
            pltpu.VMEM((_G_NCHP, _GC), jnp.int32),
            pltpu.VMEM((_GC, 384), jnp.int32),
            pltpu.VMEM((_GC, 384), jnp.int32),
            pltpu.VMEM((_GC, 384), jnp.int32),
            pltpu.VMEM((_GC, 384), jnp.int32),
            pltpu.VMEM((_GC, 384), jnp.int32),
            pltpu.VMEM((_GC, 384), jnp.int32),
            pltpu.SemaphoreType.DMA,
            pltpu.SemaphoreType.DMA,
            pltpu.SemaphoreType.DMA,
            pltpu.SemaphoreType.DMA,
            pltpu.SemaphoreType.DMA,
            pltpu.SemaphoreType.DMA,
        ],
    )
    def k(a_hbm, b_hbm, ec_hbm, en_hbm, ga_hbm, gb_hbm,
          ec_v, en_v, bufa0, bufb0, bufa1, bufb1, bufa2, bufb2,
          sg0, sg1, sg2, sw0, sw1, sw2):
        wid = lax.axis_index("s") * NC + lax.axis_index("c")
        pltpu.sync_copy(ec_hbm.at[wid], ec_v)
        pltpu.sync_copy(en_hbm.at[wid], en_v)
        base = wid * _G_EW
        bufs = ((bufa0, bufb0, sg0, sw0), (bufa1, bufb1, sg1, sw1),
                (bufa2, bufb2, sg2, sw2))

        def start_gather(i, p):
            ba, bb, sg, _ = bufs[p]
            pltpu.async_copy(a_hbm.at[ec_v.at[i]], ba, sg)
            pltpu.async_copy(b_hbm.at[en_v.at[i]], bb, sg)

        def finish(i, p):
            ba, bb, sg, sw = bufs[p]
            pltpu.make_async_copy(a_hbm.at[ec_v.at[i]], ba, sg).wait()
            pltpu.make_async_copy(b_hbm.at[en_v.at[i]], bb, sg).wait()
            pltpu.async_copy(
                ba, ga_hbm.at[pl.ds(base + i * _GC, _GC)], sw)
            pltpu.async_copy(
                bb, gb_hbm.at[pl.ds(base + i * _GC, _GC)], sw)

        def wait_write(i, p):
            ba, bb, _, sw = bufs[p]
            pltpu.make_async_copy(
                ba, ga_hbm.at[pl.ds(base + i * _GC, _GC)], sw).wait()
            pltpu.make_async_copy(
                bb, gb_hbm.at[pl.ds(base + i * _GC, _GC)], sw).wait()

        start_gather(0, 0)
        start_gather(1, 1)

        def macro_step(ii, carry):
            i = ii * 3

            def sub(q, p):
                c = i + q
                finish(c, p)

                @pl.when(c >= 1)
                def _():
                    wait_write(c - 1, (p + 2) % 3)

                @pl.when(c + 2 < _G_FULL)
                def _():
                    start_gather(c + 2, (p + 2) % 3)

            sub(0, 0)
            sub(1, 1)
            sub(2, 2)
            return carry

        lax.fori_loop(0, _G_FULL // 3, macro_step, 0)

        ba, bb, sg, sw = bufs[0]
        pltpu.async_copy(a_hbm.at[ec_v.at[_G_FULL]], ba, sg)
        pltpu.async_copy(b_hbm.at[en_v.at[_G_FULL]], bb, sg)
        pltpu.make_async_copy(a_hbm.at[ec_v.at[_G_FULL]], ba, sg).wait()
        pltpu.make_async_copy(b_hbm.at[en_v.at[_G_FULL]], bb, sg).wait()
        pltpu.sync_copy(
            ba.at[pl.ds(0, _G_TAIL)],
            ga_hbm.at[pl.ds(base + _G_FULL * _GC, _G_TAIL)])
        pltpu.sync_copy(
            bb.at[pl.ds(0, _G_TAIL)],
            gb_hbm.at[pl.ds(base + _G_FULL * _GC, _G_TAIL)])
        wait_write(_G_FULL - 1, 2)

    return k(a, b, ec3d, en3d)


def _main_body(ga_ref, gb_ref, ef_ref, lat_ref, ge_ref, be_ref, w2_ref,
               btp_ref, wpost_ref, bvec_ref, wenv_ref, benv_ref, exp_ref,
               em_ref, wt_ref):
    ef = ef_ref[...]
    g = ge_ref[...]
    beta = be_ref[...]
    scal = ef[:, :64]
    mu = jnp.mean(scal, axis=-1, keepdims=True)
    var = jnp.mean((scal - mu) * (scal - mu), axis=-1, keepdims=True)
    scal_n = (scal - mu) * lax.rsqrt(var + EPS) * g[:, :64] + beta
    b1 = ef[:, 64:160]
    b2 = ef[:, 160:240]
    bal = 0.5 * (jnp.mean(b1 * b1, axis=-1) + jnp.mean(b2 * b2, axis=-1))
    rest_n = ef[:, 64:240] * lax.rsqrt(bal + EPS)[:, None] * g[:, 64:]
    ne = jnp.concatenate([scal_n, rest_n], axis=-1)

    a_lo, a_hi = _unpack576(ga_ref[:, :288])
    b_lo, b_hi = _unpack576(gb_ref[:, :288])
    ne_dot = jnp.dot(ne.astype(jnp.bfloat16), w2_ref[...],
                     preferred_element_type=jnp.float32)
    msg_l = a_lo + b_lo + btp_ref[:, :288] + ne_dot[:, :288]
    msg_h = a_hi + b_hi + btp_ref[:, 288:] + ne_dot[:, 288:]

    scal_m = msg_l[:, :128]
    silu = scal_m * jax.nn.sigmoid(scal_m)
    gates = jax.nn.sigmoid(msg_l[:, 128:224]).astype(jnp.bfloat16)
    rest_act_l = msg_l[:, 224:288] * jnp.dot(
        gates, exp_ref[:, :64], preferred_element_type=jnp.float32)
    rest_act_h = msg_h * jnp.dot(
        gates, exp_ref[:, 64:], preferred_element_type=jnp.float32)

    em = (jnp.dot(silu.astype(jnp.bfloat16), wpost_ref[:128, :],
                  preferred_element_type=jnp.float32)
          + jnp.dot(rest_act_l.astype(jnp.bfloat16), wpost_ref[128:192, :],
                    preferred_element_type=jnp.float32)
          + jnp.dot(rest_act_h.astype(jnp.bfloat16), wpost_ref[192:, :],
                    preferred_element_type=jnp.float32)
          + bvec_ref[...])
    em_ref[...] = em

    w = (jnp.dot(lat_ref[...].astype(jnp.bfloat16), wenv_ref[...],
                 preferred_element_type=jnp.float32) + benv_ref[...])
    wexp_rest = jnp.dot(w[:, 128:224].astype(jnp.bfloat16), exp_ref[...],
                        preferred_element_type=jnp.float32)
    wt_ref[:, :128] = (POST_SCALE * em[:, :128]) * w[:, :128]
    wt_ref[:, 128:480] = (POST_SCALE * em[:, 128:480]) * wexp_rest
    wt_ref[:, 480:512] = jnp.zeros_like(wt_ref[:, 480:512])


def _main(ga, gb, ef, lat, gamma_e, beta_e, w2, btp, wpost, bvec, wenv,
          benv, expm):
    blk = 1280
    grid = E // blk
    return pl.pallas_call(
        _main_body,
        grid=(grid,),
        in_specs=[
            pl.BlockSpec((blk, 384), lambda i: (i, 0)),
            pl.BlockSpec((blk, 384), lambda i: (i, 0)),
            pl.BlockSpec((blk, 240), lambda i: (i, 0)),
            pl.BlockSpec((blk, 64), lambda i: (i, 0)),
            pl.BlockSpec((1, 240), lambda i: (0, 0)),
            pl.BlockSpec((1, 64), lambda i: (0, 0)),
            pl.BlockSpec((240, 576), lambda i: (0, 0)),
            pl.BlockSpec((1, 576), lambda i: (0, 0)),
            pl.BlockSpec((480, 480), lambda i: (0, 0)),
            pl.BlockSpec((1, 480), lambda i: (0, 0)),
            pl.BlockSpec((64, 224), lambda i: (0, 0)),
            pl.BlockSpec((1, 224), lambda i: (0, 0)),
            pl.BlockSpec((96, 352), lambda i: (0, 0)),
        ],
        out_specs=[
            pl.BlockSpec((blk, 480), lambda i: (i, 0)),
            pl.BlockSpec((blk, 512), lambda i: (i, 0)),
        ],
        out_shape=[
            jax.ShapeDtypeStruct((E, 480), jnp.float32),
            jax.ShapeDtypeStruct((E, 512), jnp.float32),
        ],
    )(ga, gb, ef, lat, gamma_e, beta_e, w2, btp, wpost, bvec, wenv, benv,
      expm)


_SC_EC = 80
_S_EW = E // NS
_S_NCH = _S_EW // _SC_EC
_S_RT = N_PAD // NS


def _sc_scatter(wt, ec3d, res):
    mesh = plsc.VectorSubcoreMesh(core_axis_name="c", subcore_axis_name="s")

    @functools.partial(
        pl.kernel,
        out_type=jax.ShapeDtypeStruct((N_PAD, 512), jnp.float32),
        mesh=mesh,
        scratch_types=[
            pltpu.VMEM((_S_NCH, _SC_EC), jnp.int32),
            pltpu.VMEM((_SC_EC, 128), jnp.float32),
            pltpu.VMEM((_SC_EC, 128), jnp.float32),
            pltpu.VMEM_SHARED((N_PAD, 128), jnp.float32),
            pltpu.SemaphoreType.DMA,
            pltpu.SemaphoreType.DMA,
        ],
    )
    def k(wt_hbm, ec_hbm, res_hbm, out_hbm, ec_v, upd0, upd1, acc, sr0, sr1):
        c = lax.axis_index("c")
        s = lax.axis_index("s")
        pltpu.sync_copy(ec_hbm.at[s], ec_v)
        r0 = s * _S_RT
        upds = ((upd0, sr0), (upd1, sr1))

        for chunk in range(2):
            col0 = (c * 2 + chunk) * 128
            pltpu.sync_copy(
                res_hbm.at[pl.ds(r0, _S_RT), pl.ds(col0, 128)],
                acc.at[pl.ds(r0, _S_RT)])
            plsc.subcore_barrier()

            def start_read(i, p):
                u, sr = upds[p]
                pltpu.async_copy(
                    wt_hbm.at[pl.ds(s * _S_EW + i * _SC_EC, _SC_EC),
                              pl.ds(col0, 128)], u, sr)

            def do_add(i, p):
                u, sr = upds[p]
                pltpu.make_async_copy(
                    wt_hbm.at[pl.ds(s * _S_EW + i * _SC_EC, _SC_EC),
                              pl.ds(col0, 128)], u, sr).wait()
                pltpu.sync_copy(u, acc.at[ec_v.at[i]], add=True)

            start_read(0, 0)
            start_read(1, 1)

            def step(ii, carry):
                i = ii * 2
                do_add(i, 0)

                @pl.when(ii + 1 < _S_NCH // 2)
                def _():
                    start_read(i + 2, 0)
                do_add(i + 1, 1)

                @pl.when(ii + 1 < _S_NCH // 2)
                def _():
                    start_read(i + 3, 1)
                return carry

            lax.fori_loop(0, _S_NCH // 2, step, 0)
            start_read(_S_NCH - 1, 0)
            do_add(_S_NCH - 1, 0)
            plsc.subcore_barrier()

            pltpu.sync_copy(
                acc.at[pl.ds(r0, _S_RT)],
                out_hbm.at[pl.ds(r0, _S_RT), pl.ds(col0, 128)])
            plsc.subcore_barrier()

    return k(wt, ec3d, res)


def _slice_body(i_ref, o_ref):
    o_ref[...] = i_ref[:, :480]


def _slice_out(x):
    blk = 1000
    return pl.pallas_call(
        _slice_body,
        grid=(N // blk,),
        in_specs=[pl.BlockSpec((blk, 512), lambda i: (i, 0))],
        out_specs=pl.BlockSpec((blk, 480), lambda i: (i, 0)),
        out_shape=jax.ShapeDtypeStruct((N, 480), jnp.float32),
    )(x)


def kernel(latents, node_features, edge_features, atom_type, edge_index,
           edge_vector, active_edges, wigner_D_all, gamma_n, beta_n,
           gamma_e, beta_e, W_tp, b_tp, W_post, b_post, W_env, b_env):
    del atom_type, edge_vector, active_edges
    ec = edge_index[0].astype(jnp.int32)
    en = edge_index[1].astype(jnp.int32)

    w1 = W_tp[:480].astype(jnp.bfloat16)
    w2 = W_tp[480:720].astype(jnp.bfloat16)
    w3 = W_tp[720:].astype(jnp.bfloat16)

    a, b, res = _node_prep(node_features, gamma_n.reshape(1, 480),
                           beta_n.reshape(1, 128), w1, w3)

    ec_g = jnp.pad(ec.reshape(NW, _G_EW),
                   ((0, 0), (0, _G_NCHP * _GC - _G_EW)))
    en_g = jnp.pad(en.reshape(NW, _G_EW),
                   ((0, 0), (0, _G_NCHP * _GC - _G_EW)))
    ga, gb = _sc_gather(a, b, ec_g.reshape(NW, _G_NCHP, _GC),
                        en_g.reshape(NW, _G_NCHP, _GC))

    bvec = jnp.concatenate([b_post, jnp.zeros((352,), jnp.float32)])
    em, wt = _main(ga, gb, edge_features, latents,
                   gamma_e.reshape(1, 240), beta_e.reshape(1, 64), w2,
                   b_tp.reshape(1, 576), W_post.astype(jnp.bfloat16),
                   bvec.reshape(1, 480), W_env.astype(jnp.bfloat16),
                   b_env.reshape(1, 224),
                   jnp.asarray(_EXP_NP, dtype=jnp.bfloat16))

    out_pad = _sc_scatter(wt, ec.reshape(NS, _S_NCH, _SC_EC), res)
    out = _slice_out(out_pad)
    return (out, em, wigner_D_all)
